# baseline (speedup 1.0000x reference)
import jax
import jax.numpy as jnp
from jax.experimental import pallas as pl


def _div_body(num_ref, mass_ref, out_ref):
    out_ref[...] = num_ref[...] / jnp.clip(mass_ref[...], 1e-08, None)


def kernel(x, pool_rows, pool_cols, pool_values, W1, b1, W2, b2):
    N, C = x.shape
    M = 50000
    h = jax.nn.gelu(x @ W1 + b1, approximate=False)
    gates = jax.nn.sigmoid((h @ W2 + b2).squeeze(-1))
    vals = pool_values * jnp.take(gates, pool_cols, axis=0)
    mass = jnp.zeros((M,), dtype=x.dtype).at[pool_rows].add(vals)
    num = jnp.zeros((M, C), dtype=x.dtype).at[pool_rows].add(
        vals[:, None] * jnp.take(x, pool_cols, axis=0))
    pooled = pl.pallas_call(
        _div_body,
        out_shape=jax.ShapeDtypeStruct((M, C), x.dtype),
        grid=(M // 1000,),
        in_specs=[
            pl.BlockSpec((1000, C), lambda i: (i, 0)),
            pl.BlockSpec((1000, 1), lambda i: (i, 0)),
        ],
        out_specs=pl.BlockSpec((1000, C), lambda i: (i, 0)),
    )(num, mass[:, None])
    return pooled


# speedup vs baseline: 9.2391x; 9.2391x over previous
import functools

import jax
import jax.numpy as jnp
from jax import lax
from jax.experimental import pallas as pl
from jax.experimental.pallas import tpu as pltpu
from jax.experimental.pallas import tpu_sc as plsc

N = 100000
M = 50000
C = 128
H = 64
EPS = 1e-08
NNZ = 800000

YW = 144
NR = 6
RROWS = 8448
ACCROWS = 8704
TROWS = 544
CHUNK = 2000
GB = 128
BN = 2000


def _erf(z):
    s = jnp.sign(z)
    a = jnp.abs(z)
    t = 1.0 / (1.0 + 0.3275911 * a)
    poly = t * (0.254829592 + t * (-0.284496736 + t * (1.421413741
           + t * (-1.453152027 + t * 1.061405429))))
    return s * (1.0 - poly * jnp.exp(-a * a))


def _gate_body(x_ref, w1_ref, b1_ref, w2_ref, b2_ref, out_ref):
    xb = x_ref[...]
    h = jnp.dot(xb, w1_ref[...], preferred_element_type=jnp.float32) + b1_ref[...]
    h = 0.5 * h * (1.0 + _erf(h * 0.7071067811865476))
    t = jnp.sum(h * w2_ref[...], axis=1, keepdims=True) + b2_ref[...]
    g = 1.0 / (1.0 + jnp.exp(-t))
    out_ref[:, :C] = xb * g
    lane = lax.broadcasted_iota(jnp.int32, (BN, YW - C), 1)
    out_ref[:, C:] = jnp.where(lane == 0, g, 0.0)


def _gate_rows(x, W1, b1, W2, b2):
    return pl.pallas_call(
        _gate_body,
        out_shape=jax.ShapeDtypeStruct((N, YW), jnp.float32),
        grid=(N // BN,),
        in_specs=[
            pl.BlockSpec((BN, C), lambda i: (i, 0)),
            pl.BlockSpec((C, H), lambda i: (0, 0)),
            pl.BlockSpec((1, H), lambda i: (0, 0)),
            pl.BlockSpec((1, H), lambda i: (0, 0)),
            pl.BlockSpec((1, 1), lambda i: (0, 0)),
        ],
        out_specs=pl.BlockSpec((BN, YW), lambda i: (i, 0)),
    )(x, W1, b1.reshape(1, H), W2.reshape(1, H), b2.reshape(1, 1))


def _pool_body(y2_hbm, rows_hbm, cols_hbm, vals_hbm, out_hbm,
               rows_v, cols_v, vals_v, ccomp, lcomp, vcomp,
               colb, lrowb, valb, stage, zbuf, wchunk, outchunk,
               acc, sem):
    c = lax.axis_index("c")
    s = lax.axis_index("s")
    nnz_base = s * (NNZ // 16)

    dump_lr = RROWS + lax.iota(jnp.int32, 16) * 16
    dump_col = lax.iota(jnp.int32, 16) * 4096
    zero16 = jnp.zeros((16,), jnp.float32)

    def _zrow(j, _):
        for u in range(YW // 16):
            zbuf[j, pl.ds(u * 16, 16)] = zero16
        return 0
    lax.fori_loop(0, 16, _zrow, 0)

    for q in range(NR // 2):
        rng = c * (NR // 2) + q
        qlo = rng * RROWS
        qsize = jnp.where(rng == NR - 1, M - (NR - 1) * RROWS, RROWS)

        def _zero(g, _):
            pltpu.sync_copy(zbuf, acc.at[pl.ds(s * TROWS + g * 16, 16)])
            return 0
        lax.fori_loop(0, TROWS // 16, _zero, 0)
        plsc.subcore_barrier()

        def _chunk(t, _):
            base = nnz_base + t * CHUNK
            pltpu.sync_copy(rows_hbm.at[pl.ds(base, CHUNK)], rows_v)
            pltpu.sync_copy(cols_hbm.at[pl.ds(base, CHUNK)], cols_v)
            pltpu.sync_copy(vals_hbm.at[pl.ds(base, CHUNK)], vals_v)

            def _filt(i, off):
                r = rows_v[pl.ds(i * 16, 16)]
                cc = cols_v[pl.ds(i * 16, 16)]
                vv = vals_v[pl.ds(i * 16, 16)]
                m = jnp.logical_and(r >= qlo, r < qlo + qsize)
                inc = plsc.cumsum(m.astype(jnp.int32))
                pos = off + inc - 1
                plsc.store_scatter(ccomp, [pos], cc, mask=m)
                plsc.store_scatter(lcomp, [pos], r - qlo, mask=m)
                plsc.store_scatter(vcomp, [pos], vv, mask=m)
                return off + inc[15]
            n = lax.fori_loop(0, CHUNK // 16, _filt, jnp.int32(0))

            for u in range(GB // 16):
                ccomp[pl.ds(n + u * 16, 16)] = dump_col
                lcomp[pl.ds(n + u * 16, 16)] = dump_lr
                vcomp[pl.ds(n + u * 16, 16)] = zero16

            def _batch(b, _):
                boff = b * GB
                for u in range(GB // 16):
                    colb[pl.ds(u * 16, 16)] = ccomp[pl.ds(boff + u * 16, 16)]
                    lrowb[pl.ds(u * 16, 16)] = lcomp[pl.ds(boff + u * 16, 16)]
                    valb[pl.ds(u * 16, 16)] = vcomp[pl.ds(boff + u * 16, 16)]
                pltpu.async_copy(y2_hbm.at[colb], stage, sem).wait()

                def _scale(g, _):
                    vv16 = valb[pl.ds(g * 16, 16)]
                    for jj in range(16):
                        j = g * 16 + jj
                        v = vv16[jj]
                        for u in range(YW // 16):
                            stage[j, pl.ds(u * 16, 16)] = stage[j, pl.ds(u * 16, 16)] * v
                    return 0
                lax.fori_loop(0, GB // 16, _scale, 0)
                pltpu.sync_copy(stage, acc.at[lrowb], add=True)
                return 0
            nb = (n + GB - 1) // GB
            lax.fori_loop(0, nb, _batch, 0)
            return 0
        lax.fori_loop(0, NNZ // 16 // CHUNK, _chunk, 0)
        plsc.subcore_barrier()

        def _wb(g, _):
            lrow0 = s * TROWS + g * 16
            pltpu.sync_copy(acc.at[pl.ds(lrow0, 16)], wchunk)

            masses = plsc.load_gather(
                wchunk, [lax.iota(jnp.int32, 16),
                         jnp.full((16,), C, jnp.int32)])
            denom = jnp.maximum(masses, EPS)
            for jj in range(16):
                dv = denom[jj]
                for u in range(C // 16):
                    outchunk[jj, pl.ds(u * 16, 16)] = wchunk[jj, pl.ds(u * 16, 16)] / dv

            grow = qlo + lrow0

            @pl.when(lrow0 + 16 <= qsize)
            def _full():
                pltpu.sync_copy(outchunk, out_hbm.at[pl.ds(grow, 16)])
            return 0
        lax.fori_loop(0, TROWS // 16, _wb, 0)
        plsc.subcore_barrier()


def _pool_sc(y2, pool_rows, pool_cols, pool_values):
    mesh = plsc.VectorSubcoreMesh(core_axis_name="c", subcore_axis_name="s")
    f = functools.partial(
        pl.kernel, _pool_body, mesh=mesh,
        compiler_params=pltpu.CompilerParams(
            needs_layout_passes=False, use_tc_tiling_on_sc=False),
        out_type=jax.ShapeDtypeStruct((M, C), jnp.float32),
        scratch_types=[
            pltpu.VMEM((CHUNK,), jnp.int32),
            pltpu.VMEM((CHUNK,), jnp.int32),
            pltpu.VMEM((CHUNK,), jnp.float32),
            pltpu.VMEM((CHUNK + GB,), jnp.int32),
            pltpu.VMEM((CHUNK + GB,), jnp.int32),
            pltpu.VMEM((CHUNK + GB,), jnp.float32),
            pltpu.VMEM((GB,), jnp.int32),
            pltpu.VMEM((GB,), jnp.int32),
            pltpu.VMEM((GB,), jnp.float32),
            pltpu.VMEM((GB, YW), jnp.float32),
            pltpu.VMEM((16, YW), jnp.float32),
            pltpu.VMEM((16, YW), jnp.float32),
            pltpu.VMEM((16, C), jnp.float32),
            pltpu.VMEM_SHARED((ACCROWS, YW), jnp.float32),
            pltpu.SemaphoreType.DMA,
        ],
    )()
    return f(y2, pool_rows, pool_cols, pool_values)


def kernel(x, pool_rows, pool_cols, pool_values, W1, b1, W2, b2):
    y2 = _gate_rows(x, W1, b1, W2, b2)
    return _pool_sc(y2, pool_rows, pool_cols, pool_values)


# speedup vs baseline: 10.4518x; 1.1313x over previous
import functools

import jax
import jax.numpy as jnp
from jax import lax
from jax.experimental import pallas as pl
from jax.experimental.pallas import tpu as pltpu
from jax.experimental.pallas import tpu_sc as plsc

N = 100000
M = 50000
C = 128
H = 64
EPS = 1e-08
NNZ = 800000

YW = 144
NR = 6
RROWS = 8448
ACCROWS = 8704
TROWS = 544
CHUNK = 2000
GB = 96
BN = 2000


def _erf(z):
    s = jnp.sign(z)
    a = jnp.abs(z)
    t = 1.0 / (1.0 + 0.3275911 * a)
    poly = t * (0.254829592 + t * (-0.284496736 + t * (1.421413741
           + t * (-1.453152027 + t * 1.061405429))))
    return s * (1.0 - poly * jnp.exp(-a * a))


def _gate_body(x_ref, w1_ref, b1_ref, w2_ref, b2_ref, out_ref):
    xb = x_ref[...]
    h = jnp.dot(xb, w1_ref[...], preferred_element_type=jnp.float32) + b1_ref[...]
    h = 0.5 * h * (1.0 + _erf(h * 0.7071067811865476))
    t = jnp.sum(h * w2_ref[...], axis=1, keepdims=True) + b2_ref[...]
    g = 1.0 / (1.0 + jnp.exp(-t))
    out_ref[:, :C] = xb * g
    lane = lax.broadcasted_iota(jnp.int32, (BN, YW - C), 1)
    out_ref[:, C:] = jnp.where(lane == 0, g, 0.0)


def _gate_rows(x, W1, b1, W2, b2):
    return pl.pallas_call(
        _gate_body,
        out_shape=jax.ShapeDtypeStruct((N, YW), jnp.float32),
        grid=(N // BN,),
        in_specs=[
            pl.BlockSpec((BN, C), lambda i: (i, 0)),
            pl.BlockSpec((C, H), lambda i: (0, 0)),
            pl.BlockSpec((1, H), lambda i: (0, 0)),
            pl.BlockSpec((1, H), lambda i: (0, 0)),
            pl.BlockSpec((1, 1), lambda i: (0, 0)),
        ],
        out_specs=pl.BlockSpec((BN, YW), lambda i: (i, 0)),
    )(x, W1, b1.reshape(1, H), W2.reshape(1, H), b2.reshape(1, 1))


def _pool_body(y2_hbm, rows_hbm, cols_hbm, vals_hbm, out_hbm,
               rows_v, cols_v, vals_v, ccomp, lcomp, vcomp,
               colb, lrowb, valb, stage, colb2, lrowb2, valb2, stage2,
               zbuf, wchunk, outchunk,
               acc, gsem, gsem2, ssem, ssem2):
    c = lax.axis_index("c")
    s = lax.axis_index("s")
    nnz_base = s * (NNZ // 16)

    dump_lr = RROWS + lax.iota(jnp.int32, 16) * 16
    dump_col = lax.iota(jnp.int32, 16) * 4096
    zero16 = jnp.zeros((16,), jnp.float32)

    def _zrow(j, _):
        for u in range(YW // 16):
            zbuf[j, pl.ds(u * 16, 16)] = zero16
        return 0
    lax.fori_loop(0, 16, _zrow, 0)

    for q in range(NR // 2):
        rng = c * (NR // 2) + q
        qlo = rng * RROWS
        qsize = jnp.where(rng == NR - 1, M - (NR - 1) * RROWS, RROWS)

        def _zero(g, _):
            pltpu.sync_copy(zbuf, acc.at[pl.ds(s * TROWS + g * 16, 16)])
            return 0
        lax.fori_loop(0, TROWS // 16, _zero, 0)
        plsc.subcore_barrier()

        def _chunk(t, _):
            base = nnz_base + t * CHUNK
            pltpu.sync_copy(rows_hbm.at[pl.ds(base, CHUNK)], rows_v)
            pltpu.sync_copy(cols_hbm.at[pl.ds(base, CHUNK)], cols_v)
            pltpu.sync_copy(vals_hbm.at[pl.ds(base, CHUNK)], vals_v)

            def _filt(i, off):
                r = rows_v[pl.ds(i * 16, 16)]
                cc = cols_v[pl.ds(i * 16, 16)]
                vv = vals_v[pl.ds(i * 16, 16)]
                m = jnp.logical_and(r >= qlo, r < qlo + qsize)
                inc = plsc.cumsum(m.astype(jnp.int32))
                pos = off + inc - 1
                plsc.store_scatter(ccomp, [pos], cc, mask=m)
                plsc.store_scatter(lcomp, [pos], r - qlo, mask=m)
                plsc.store_scatter(vcomp, [pos], vv, mask=m)
                return off + inc[15]
            n = lax.fori_loop(0, CHUNK // 16, _filt, jnp.int32(0))

            for u in range(2 * GB // 16):
                ccomp[pl.ds(n + u * 16, 16)] = dump_col
                lcomp[pl.ds(n + u * 16, 16)] = dump_lr
                vcomp[pl.ds(n + u * 16, 16)] = zero16

            npairs = jnp.maximum((n + 2 * GB - 1) // (2 * GB), 1)
            colbs = (colb, colb2)
            lrowbs = (lrowb, lrowb2)
            valbs = (valb, valb2)
            stages = (stage, stage2)
            gsems = (gsem, gsem2)

            def _load_idx(bi, par):
                boff = bi * GB
                for u in range(GB // 16):
                    colbs[par][pl.ds(u * 16, 16)] = ccomp[pl.ds(boff + u * 16, 16)]
                    lrowbs[par][pl.ds(u * 16, 16)] = lcomp[pl.ds(boff + u * 16, 16)]
                    valbs[par][pl.ds(u * 16, 16)] = vcomp[pl.ds(boff + u * 16, 16)]

            def _scale_scatter(par):
                def _scale(g, _):
                    vv16 = valbs[par][pl.ds(g * 16, 16)]
                    for jj in range(16):
                        j = g * 16 + jj
                        v = vv16[jj]
                        for u in range(YW // 16):
                            stages[par][j, pl.ds(u * 16, 16)] = (
                                stages[par][j, pl.ds(u * 16, 16)] * v)
                    return 0
                lax.fori_loop(0, GB // 16, _scale, 0)
                pltpu.sync_copy(stages[par], acc.at[lrowbs[par]], add=True)

            _load_idx(jnp.int32(0), 0)
            pltpu.async_copy(y2_hbm.at[colbs[0]], stages[0], gsems[0])

            def _pairk(k, _):
                _load_idx(k * 2 + 1, 1)
                pltpu.async_copy(y2_hbm.at[colbs[1]], stages[1], gsems[1])
                pltpu.make_async_copy(
                    y2_hbm.at[colbs[0]], stages[0], gsems[0]).wait()
                _scale_scatter(0)

                @pl.when(k + 1 < npairs)
                def _pre():
                    _load_idx(k * 2 + 2, 0)
                    pltpu.async_copy(y2_hbm.at[colbs[0]], stages[0], gsems[0])
                pltpu.make_async_copy(
                    y2_hbm.at[colbs[1]], stages[1], gsems[1]).wait()
                _scale_scatter(1)
                return 0
            lax.fori_loop(0, npairs, _pairk, 0)
            return 0
        lax.fori_loop(0, NNZ // 16 // CHUNK, _chunk, 0)
        plsc.subcore_barrier()

        def _wb(g, _):
            lrow0 = s * TROWS + g * 16
            pltpu.sync_copy(acc.at[pl.ds(lrow0, 16)], wchunk)

            masses = plsc.load_gather(
                wchunk, [lax.iota(jnp.int32, 16),
                         jnp.full((16,), C, jnp.int32)])
            denom = jnp.maximum(masses, EPS)
            for jj in range(16):
                dv = denom[jj]
                for u in range(C // 16):
                    outchunk[jj, pl.ds(u * 16, 16)] = wchunk[jj, pl.ds(u * 16, 16)] / dv

            grow = qlo + lrow0

            @pl.when(lrow0 + 16 <= qsize)
            def _full():
                pltpu.sync_copy(outchunk, out_hbm.at[pl.ds(grow, 16)])
            return 0
        lax.fori_loop(0, TROWS // 16, _wb, 0)
        plsc.subcore_barrier()


def _pool_sc(y2, pool_rows, pool_cols, pool_values):
    mesh = plsc.VectorSubcoreMesh(core_axis_name="c", subcore_axis_name="s")
    f = functools.partial(
        pl.kernel, _pool_body, mesh=mesh,
        compiler_params=pltpu.CompilerParams(
            needs_layout_passes=False, use_tc_tiling_on_sc=False),
        out_type=jax.ShapeDtypeStruct((M, C), jnp.float32),
        scratch_types=[
            pltpu.VMEM((CHUNK,), jnp.int32),
            pltpu.VMEM((CHUNK,), jnp.int32),
            pltpu.VMEM((CHUNK,), jnp.float32),
            pltpu.VMEM((CHUNK + GB,), jnp.int32),
            pltpu.VMEM((CHUNK + GB,), jnp.int32),
            pltpu.VMEM((CHUNK + GB,), jnp.float32),
            pltpu.VMEM((GB,), jnp.int32),
            pltpu.VMEM((GB,), jnp.int32),
            pltpu.VMEM((GB,), jnp.float32),
            pltpu.VMEM((GB, YW), jnp.float32),
            pltpu.VMEM((GB,), jnp.int32),
            pltpu.VMEM((GB,), jnp.int32),
            pltpu.VMEM((GB,), jnp.float32),
            pltpu.VMEM((GB, YW), jnp.float32),
            pltpu.VMEM((16, YW), jnp.float32),
            pltpu.VMEM((16, YW), jnp.float32),
            pltpu.VMEM((16, C), jnp.float32),
            pltpu.VMEM_SHARED((ACCROWS, YW), jnp.float32),
            pltpu.SemaphoreType.DMA,
            pltpu.SemaphoreType.DMA,
            pltpu.SemaphoreType.DMA,
            pltpu.SemaphoreType.DMA,
        ],
    )()
    return f(y2, pool_rows, pool_cols, pool_values)


def kernel(x, pool_rows, pool_cols, pool_values, W1, b1, W2, b2):
    y2 = _gate_rows(x, W1, b1, W2, b2)
    return _pool_sc(y2, pool_rows, pool_cols, pool_values)


# speedup vs baseline: 10.4945x; 1.0041x over previous
import functools

import jax
import jax.numpy as jnp
from jax import lax
from jax.experimental import pallas as pl
from jax.experimental.pallas import tpu as pltpu
from jax.experimental.pallas import tpu_sc as plsc

N = 100000
M = 50000
C = 128
H = 64
EPS = 1e-08
NNZ = 800000

YW = 144
NR = 6
RROWS = 8448
ACCROWS = 8704
TROWS = 544
CHUNK = 2000
GB = 96
BN = 2000


def _erf(z):
    s = jnp.sign(z)
    a = jnp.abs(z)
    t = 1.0 / (1.0 + 0.3275911 * a)
    poly = t * (0.254829592 + t * (-0.284496736 + t * (1.421413741
           + t * (-1.453152027 + t * 1.061405429))))
    return s * (1.0 - poly * jnp.exp(-a * a))


def _gate_body(x_ref, w1_ref, b1_ref, w2_ref, b2_ref, out_ref):
    xb = x_ref[...]
    h = jnp.dot(xb, w1_ref[...], preferred_element_type=jnp.float32) + b1_ref[...]
    h = 0.5 * h * (1.0 + _erf(h * 0.7071067811865476))
    t = jnp.sum(h * w2_ref[...], axis=1, keepdims=True) + b2_ref[...]
    g = 1.0 / (1.0 + jnp.exp(-t))
    out_ref[:, :C] = xb * g
    lane = lax.broadcasted_iota(jnp.int32, (BN, YW - C), 1)
    out_ref[:, C:] = jnp.where(lane == 0, g, 0.0)


def _gate_rows(x, W1, b1, W2, b2):
    return pl.pallas_call(
        _gate_body,
        out_shape=jax.ShapeDtypeStruct((N, YW), jnp.float32),
        grid=(N // BN,),
        in_specs=[
            pl.BlockSpec((BN, C), lambda i: (i, 0)),
            pl.BlockSpec((C, H), lambda i: (0, 0)),
            pl.BlockSpec((1, H), lambda i: (0, 0)),
            pl.BlockSpec((1, H), lambda i: (0, 0)),
            pl.BlockSpec((1, 1), lambda i: (0, 0)),
        ],
        out_specs=pl.BlockSpec((BN, YW), lambda i: (i, 0)),
    )(x, W1, b1.reshape(1, H), W2.reshape(1, H), b2.reshape(1, 1))


def _pool_body(y2_hbm, rows_hbm, cols_hbm, vals_hbm, out_hbm,
               rows_v, cols_v, vals_v, ccomp, lcomp, vcomp,
               colb, lrowb, valb, stage, colb2, lrowb2, valb2, stage2,
               zbuf, wchunk, outchunk,
               acc, gsem, gsem2, ssem, ssem2):
    c = lax.axis_index("c")
    s = lax.axis_index("s")
    nnz_base = s * (NNZ // 16)

    dump_lr = RROWS + lax.iota(jnp.int32, 16) * 16
    dump_col = lax.iota(jnp.int32, 16) * 4096
    zero16 = jnp.zeros((16,), jnp.float32)

    def _zrow(j, _):
        for u in range(YW // 16):
            zbuf[j, pl.ds(u * 16, 16)] = zero16
        return 0
    lax.fori_loop(0, 16, _zrow, 0)

    for q in range(NR // 2):
        rng = c * (NR // 2) + q
        qlo = rng * RROWS
        qsize = jnp.where(rng == NR - 1, M - (NR - 1) * RROWS, RROWS)

        def _zero(g, _):
            pltpu.sync_copy(zbuf, acc.at[pl.ds(s * TROWS + g * 16, 16)])
            return 0
        lax.fori_loop(0, TROWS // 16, _zero, 0)
        plsc.subcore_barrier()

        def _chunk(t, _):
            base = nnz_base + t * CHUNK
            pltpu.sync_copy(rows_hbm.at[pl.ds(base, CHUNK)], rows_v)
            pltpu.sync_copy(cols_hbm.at[pl.ds(base, CHUNK)], cols_v)
            pltpu.sync_copy(vals_hbm.at[pl.ds(base, CHUNK)], vals_v)

            def _filt(i, off):
                r = rows_v[pl.ds(i * 16, 16)]
                cc = cols_v[pl.ds(i * 16, 16)]
                vv = vals_v[pl.ds(i * 16, 16)]
                m = jnp.logical_and(r >= qlo, r < qlo + qsize)
                inc = plsc.cumsum(m.astype(jnp.int32))
                pos = off + inc - 1
                plsc.store_scatter(ccomp, [pos], cc, mask=m)
                plsc.store_scatter(lcomp, [pos], r - qlo, mask=m)
                plsc.store_scatter(vcomp, [pos], vv, mask=m)
                return off + inc[15]
            n = lax.fori_loop(0, CHUNK // 16, _filt, jnp.int32(0))

            for u in range(2 * GB // 16):
                ccomp[pl.ds(n + u * 16, 16)] = dump_col
                lcomp[pl.ds(n + u * 16, 16)] = dump_lr
                vcomp[pl.ds(n + u * 16, 16)] = zero16

            npairs = jnp.maximum((n + 2 * GB - 1) // (2 * GB), 1)
            colbs = (colb, colb2)
            lrowbs = (lrowb, lrowb2)
            valbs = (valb, valb2)
            stages = (stage, stage2)
            gsems = (gsem, gsem2)
            ssems = (ssem, ssem2)

            def _load_idx(bi, par):
                boff = bi * GB
                for u in range(GB // 16):
                    colbs[par][pl.ds(u * 16, 16)] = ccomp[pl.ds(boff + u * 16, 16)]
                    lrowbs[par][pl.ds(u * 16, 16)] = lcomp[pl.ds(boff + u * 16, 16)]
                    valbs[par][pl.ds(u * 16, 16)] = vcomp[pl.ds(boff + u * 16, 16)]

            def _scale_scatter(par):
                def _scale(g, _):
                    vv16 = valbs[par][pl.ds(g * 16, 16)]
                    for jj in range(16):
                        j = g * 16 + jj
                        v = vv16[jj]
                        for u in range(YW // 16):
                            stages[par][j, pl.ds(u * 16, 16)] = (
                                stages[par][j, pl.ds(u * 16, 16)] * v)
                    return 0
                lax.fori_loop(0, GB // 16, _scale, 0)
                pltpu.async_copy(stages[par], acc.at[lrowbs[par]],
                                 ssems[par], add=True)

            def _drain(par):
                pltpu.make_async_copy(
                    stages[par], acc.at[lrowbs[par]], ssems[par]).wait()

            _load_idx(jnp.int32(0), 0)
            pltpu.async_copy(y2_hbm.at[colbs[0]], stages[0], gsems[0])

            def _pairk(k, _):
                @pl.when(k > 0)
                def _d1():
                    _drain(1)
                _load_idx(k * 2 + 1, 1)
                pltpu.async_copy(y2_hbm.at[colbs[1]], stages[1], gsems[1])
                pltpu.make_async_copy(
                    y2_hbm.at[colbs[0]], stages[0], gsems[0]).wait()
                _scale_scatter(0)

                @pl.when(k + 1 < npairs)
                def _pre():
                    _drain(0)
                    _load_idx(k * 2 + 2, 0)
                    pltpu.async_copy(y2_hbm.at[colbs[0]], stages[0], gsems[0])
                pltpu.make_async_copy(
                    y2_hbm.at[colbs[1]], stages[1], gsems[1]).wait()
                _scale_scatter(1)
                return 0
            lax.fori_loop(0, npairs, _pairk, 0)
            _drain(0)
            _drain(1)
            return 0
        lax.fori_loop(0, NNZ // 16 // CHUNK, _chunk, 0)
        plsc.subcore_barrier()

        def _wb(g, _):
            lrow0 = s * TROWS + g * 16
            pltpu.sync_copy(acc.at[pl.ds(lrow0, 16)], wchunk)

            masses = plsc.load_gather(
                wchunk, [lax.iota(jnp.int32, 16),
                         jnp.full((16,), C, jnp.int32)])
            denom = jnp.maximum(masses, EPS)
            for jj in range(16):
                dv = denom[jj]
                for u in range(C // 16):
                    outchunk[jj, pl.ds(u * 16, 16)] = wchunk[jj, pl.ds(u * 16, 16)] / dv

            grow = qlo + lrow0

            @pl.when(lrow0 + 16 <= qsize)
            def _full():
                pltpu.sync_copy(outchunk, out_hbm.at[pl.ds(grow, 16)])
            return 0
        lax.fori_loop(0, TROWS // 16, _wb, 0)
        plsc.subcore_barrier()


def _pool_sc(y2, pool_rows, pool_cols, pool_values):
    mesh = plsc.VectorSubcoreMesh(core_axis_name="c", subcore_axis_name="s")
    f = functools.partial(
        pl.kernel, _pool_body, mesh=mesh,
        compiler_params=pltpu.CompilerParams(
            needs_layout_passes=False, use_tc_tiling_on_sc=False),
        out_type=jax.ShapeDtypeStruct((M, C), jnp.float32),
        scratch_types=[
            pltpu.VMEM((CHUNK,), jnp.int32),
            pltpu.VMEM((CHUNK,), jnp.int32),
            pltpu.VMEM((CHUNK,), jnp.float32),
            pltpu.VMEM((CHUNK + GB,), jnp.int32),
            pltpu.VMEM((CHUNK + GB,), jnp.int32),
            pltpu.VMEM((CHUNK + GB,), jnp.float32),
            pltpu.VMEM((GB,), jnp.int32),
            pltpu.VMEM((GB,), jnp.int32),
            pltpu.VMEM((GB,), jnp.float32),
            pltpu.VMEM((GB, YW), jnp.float32),
            pltpu.VMEM((GB,), jnp.int32),
            pltpu.VMEM((GB,), jnp.int32),
            pltpu.VMEM((GB,), jnp.float32),
            pltpu.VMEM((GB, YW), jnp.float32),
            pltpu.VMEM((16, YW), jnp.float32),
            pltpu.VMEM((16, YW), jnp.float32),
            pltpu.VMEM((16, C), jnp.float32),
            pltpu.VMEM_SHARED((ACCROWS, YW), jnp.float32),
            pltpu.SemaphoreType.DMA,
            pltpu.SemaphoreType.DMA,
            pltpu.SemaphoreType.DMA,
            pltpu.SemaphoreType.DMA,
        ],
    )()
    return f(y2, pool_rows, pool_cols, pool_values)


def kernel(x, pool_rows, pool_cols, pool_values, W1, b1, W2, b2):
    y2 = _gate_rows(x, W1, b1, W2, b2)
    return _pool_sc(y2, pool_rows, pool_cols, pool_values)
